# Initial kernel scaffold; baseline (speedup 1.0000x reference)
import jax
import jax.numpy as jnp
from jax.experimental import pallas as pl


def kernel(input, value):
    raise NotImplementedError("write your pallas kernel here")


# speedup vs baseline: 1173.9840x; 1173.9840x over previous
import functools

import jax
import jax.numpy as jnp
from jax import lax
from jax.experimental import pallas as pl
from jax.experimental.pallas import tpu as pltpu
from jax.experimental.pallas import tpu_sc as plsc

_NODES = 4096
_N = 16777216
_H = 1.0 / (_NODES - 1)

_NC = 2
_NS = 16
_NW = _NC * _NS
_LANES = 16

_PER_TILE = _N // _NW
_CHUNK = 16384
_NCHUNK = _PER_TILE // _CHUNK
_VECS = _CHUNK // _LANES


def _akima_coefs(value):
    y = value
    n = y.shape[0]
    m = (y[1:] - y[:-1]) / _H
    m_m1 = 2.0 * m[0] - m[1]
    m_m2 = 2.0 * m_m1 - m[0]
    m_p0 = 2.0 * m[-1] - m[-2]
    m_p1 = 2.0 * m_p0 - m[-1]
    mm = jnp.concatenate(
        [jnp.stack([m_m2, m_m1]), m, jnp.stack([m_p0, m_p1])]
    )
    dm = jnp.abs(mm[1:] - mm[:-1])
    w1 = dm[2:n + 2]
    w2 = dm[0:n]
    ma = mm[1:n + 1]
    mb = mm[2:n + 2]
    denom = w1 + w2
    safe = jnp.where(denom > 1e-9, denom, 1.0)
    t = jnp.where(denom > 1e-9, (w1 * ma + w2 * mb) / safe, 0.5 * (ma + mb))
    t0 = t[:-1]
    t1 = t[1:]
    c2 = (3.0 * m - 2.0 * t0 - t1) / _H
    c3 = (t0 + t1 - 2.0 * m) / (_H * _H)
    pad = jnp.zeros((1,), jnp.float32)
    return jnp.stack([
        y,
        t,
        jnp.concatenate([c2, pad]),
        jnp.concatenate([c3, pad]),
    ])


def _sc_eval(x, coef):
    mesh = plsc.VectorSubcoreMesh(core_axis_name="c", subcore_axis_name="s")

    @functools.partial(
        pl.kernel,
        out_type=jax.ShapeDtypeStruct((_N,), jnp.float32),
        mesh=mesh,
        compiler_params=pltpu.CompilerParams(needs_layout_passes=False),
        scratch_types=[
            pltpu.VMEM((_NODES,), jnp.float32),
            pltpu.VMEM((_NODES,), jnp.float32),
            pltpu.VMEM((_NODES,), jnp.float32),
            pltpu.VMEM((_NODES,), jnp.float32),
            pltpu.VMEM((_CHUNK,), jnp.float32),
            pltpu.VMEM((_CHUNK,), jnp.float32),
        ],
    )
    def body(x_hbm, coef_hbm, out_hbm, tab0, tab1, tab2, tab3, xin, res):
        cid = lax.axis_index("c")
        sid = lax.axis_index("s")
        wid = sid * _NC + cid
        base = wid * _PER_TILE

        pltpu.sync_copy(coef_hbm.at[0], tab0)
        pltpu.sync_copy(coef_hbm.at[1], tab1)
        pltpu.sync_copy(coef_hbm.at[2], tab2)
        pltpu.sync_copy(coef_hbm.at[3], tab3)

        def chunk(c, _):
            off = base + c * _CHUNK
            pltpu.sync_copy(x_hbm.at[pl.ds(off, _CHUNK)], xin)

            def vec(i, _):
                xv = xin[pl.ds(i * _LANES, _LANES)]
                s = xv * jnp.float32(_NODES - 1)
                si = s.astype(jnp.int32)
                si = jnp.minimum(jnp.maximum(si, 0), _NODES - 2)
                r = xv - si.astype(jnp.float32) * jnp.float32(_H)
                c0 = plsc.load_gather(tab0, [si])
                c1 = plsc.load_gather(tab1, [si])
                c2 = plsc.load_gather(tab2, [si])
                c3 = plsc.load_gather(tab3, [si])
                res[pl.ds(i * _LANES, _LANES)] = (
                    c0 + r * (c1 + r * (c2 + r * c3))
                )
                return _

            lax.fori_loop(0, _VECS, vec, None)
            pltpu.sync_copy(res, out_hbm.at[pl.ds(off, _CHUNK)])
            return _

        lax.fori_loop(0, _NCHUNK, chunk, None)

    return body(x, coef)


@jax.jit
def kernel(input, value):
    coef = _akima_coefs(value)
    return _sc_eval(input, coef)


# speedup vs baseline: 3716.1952x; 3.1655x over previous
import functools

import jax
import jax.numpy as jnp
from jax import lax
from jax.experimental import pallas as pl
from jax.experimental.pallas import tpu as pltpu
from jax.experimental.pallas import tpu_sc as plsc

_NODES = 4096
_N = 16777216
_H = 1.0 / (_NODES - 1)

_NC = 2
_NS = 16
_NW = _NC * _NS
_LANES = 16

_PER_TILE = _N // _NW
_CHUNK = 16384
_NCHUNK = _PER_TILE // _CHUNK
_VECS = _CHUNK // _LANES


def _akima_coefs(value):
    y = value
    n = y.shape[0]
    m = (y[1:] - y[:-1]) / _H
    m_m1 = 2.0 * m[0] - m[1]
    m_m2 = 2.0 * m_m1 - m[0]
    m_p0 = 2.0 * m[-1] - m[-2]
    m_p1 = 2.0 * m_p0 - m[-1]
    mm = jnp.concatenate(
        [jnp.stack([m_m2, m_m1]), m, jnp.stack([m_p0, m_p1])]
    )
    dm = jnp.abs(mm[1:] - mm[:-1])
    w1 = dm[2:n + 2]
    w2 = dm[0:n]
    ma = mm[1:n + 1]
    mb = mm[2:n + 2]
    denom = w1 + w2
    safe = jnp.where(denom > 1e-9, denom, 1.0)
    t = jnp.where(denom > 1e-9, (w1 * ma + w2 * mb) / safe, 0.5 * (ma + mb))
    t0 = t[:-1]
    t1 = t[1:]
    c2 = (3.0 * m - 2.0 * t0 - t1) / _H
    c3 = (t0 + t1 - 2.0 * m) / (_H * _H)
    pad = jnp.zeros((1,), jnp.float32)
    return jnp.stack([
        y,
        t,
        jnp.concatenate([c2, pad]),
        jnp.concatenate([c3, pad]),
    ])


def _sc_eval(x, coef):
    mesh = plsc.VectorSubcoreMesh(core_axis_name="c", subcore_axis_name="s")

    @functools.partial(
        pl.kernel,
        out_type=jax.ShapeDtypeStruct((_N,), jnp.float32),
        mesh=mesh,
        compiler_params=pltpu.CompilerParams(needs_layout_passes=False),
        scratch_types=[
            pltpu.VMEM((_NODES,), jnp.float32),
            pltpu.VMEM((_NODES,), jnp.float32),
            pltpu.VMEM((_NODES,), jnp.float32),
            pltpu.VMEM((_NODES,), jnp.float32),
            pltpu.VMEM((_CHUNK,), jnp.float32),
            pltpu.VMEM((_CHUNK,), jnp.float32),
            pltpu.VMEM((_CHUNK,), jnp.float32),
            pltpu.VMEM((_CHUNK,), jnp.float32),
            pltpu.SemaphoreType.DMA,
            pltpu.SemaphoreType.DMA,
            pltpu.SemaphoreType.DMA,
            pltpu.SemaphoreType.DMA,
        ],
    )
    def body(x_hbm, coef_hbm, out_hbm, tab0, tab1, tab2, tab3,
             xin0, xin1, res0, res1, si0, si1, so0, so1):
        cid = lax.axis_index("c")
        sid = lax.axis_index("s")
        wid = sid * _NC + cid
        base = wid * _PER_TILE

        pltpu.sync_copy(coef_hbm.at[0], tab0)
        pltpu.sync_copy(coef_hbm.at[1], tab1)
        pltpu.sync_copy(coef_hbm.at[2], tab2)
        pltpu.sync_copy(coef_hbm.at[3], tab3)

        bufs = ((xin0, res0, si0, so0), (xin1, res1, si1, so1))
        tabs = (tab0, tab1, tab2, tab3)

        def load(c, xin, si):
            return pltpu.make_async_copy(
                x_hbm.at[pl.ds(base + c * _CHUNK, _CHUNK)], xin, si)

        def store(c, res, so):
            return pltpu.make_async_copy(
                res, out_hbm.at[pl.ds(base + c * _CHUNK, _CHUNK)], so)

        load(0, xin0, si0).start()
        load(1, xin1, si1).start()

        def outer(j, _):
            for b in range(2):
                xin, res, si, so = bufs[b]
                c = 2 * j + b
                load(c, xin, si).wait()

                @pl.when(c >= 2)
                def _drain():
                    store(c - 2, res, so).wait()

                @plsc.parallel_loop(0, _CHUNK, step=_LANES, unroll=8)
                def vec(i):
                    xv = xin[pl.ds(i, _LANES)]
                    s = xv * jnp.float32(_NODES - 1)
                    si_ = s.astype(jnp.int32)
                    si_ = jnp.minimum(jnp.maximum(si_, 0), _NODES - 2)
                    r = xv - si_.astype(jnp.float32) * jnp.float32(_H)
                    c0 = plsc.load_gather(tabs[0], [si_])
                    c1 = plsc.load_gather(tabs[1], [si_])
                    c2 = plsc.load_gather(tabs[2], [si_])
                    c3 = plsc.load_gather(tabs[3], [si_])
                    res[pl.ds(i, _LANES)] = c0 + r * (c1 + r * (c2 + r * c3))

                store(c, res, so).start()

                @pl.when(c + 2 < _NCHUNK)
                def _next():
                    load(c + 2, xin, si).start()

            return _

        lax.fori_loop(0, _NCHUNK // 2, outer, None)
        store(_NCHUNK - 2, res0, so0).wait()
        store(_NCHUNK - 1, res1, so1).wait()

    return body(x, coef)


@jax.jit
def kernel(input, value):
    coef = _akima_coefs(value)
    return _sc_eval(input, coef)


# speedup vs baseline: 3847.1772x; 1.0352x over previous
import functools

import jax
import jax.numpy as jnp
from jax import lax
from jax.experimental import pallas as pl
from jax.experimental.pallas import tpu as pltpu
from jax.experimental.pallas import tpu_sc as plsc

_NODES = 4096
_N = 16777216
_H = 1.0 / (_NODES - 1)

_NC = 2
_NS = 16
_NW = _NC * _NS
_LANES = 16

_PER_TILE = _N // _NW
_CHUNK = 16384
_NCHUNK = _PER_TILE // _CHUNK
_VECS = _CHUNK // _LANES


def _akima_coefs(value):
    y = value
    n = y.shape[0]
    m = (y[1:] - y[:-1]) / _H
    m_m1 = 2.0 * m[0] - m[1]
    m_m2 = 2.0 * m_m1 - m[0]
    m_p0 = 2.0 * m[-1] - m[-2]
    m_p1 = 2.0 * m_p0 - m[-1]
    mm = jnp.concatenate(
        [jnp.stack([m_m2, m_m1]), m, jnp.stack([m_p0, m_p1])]
    )
    dm = jnp.abs(mm[1:] - mm[:-1])
    w1 = dm[2:n + 2]
    w2 = dm[0:n]
    ma = mm[1:n + 1]
    mb = mm[2:n + 2]
    denom = w1 + w2
    safe = jnp.where(denom > 1e-9, denom, 1.0)
    t = jnp.where(denom > 1e-9, (w1 * ma + w2 * mb) / safe, 0.5 * (ma + mb))
    t0 = t[:-1]
    t1 = t[1:]
    c2 = (3.0 * m - 2.0 * t0 - t1) / _H
    c3 = (t0 + t1 - 2.0 * m) / (_H * _H)
    pad = jnp.zeros((1,), jnp.float32)
    return jnp.stack([
        y,
        t,
        jnp.concatenate([c2, pad]),
        jnp.concatenate([c3, pad]),
    ])


def _sc_eval(x, coef):
    mesh = plsc.VectorSubcoreMesh(core_axis_name="c", subcore_axis_name="s")

    @functools.partial(
        pl.kernel,
        out_type=jax.ShapeDtypeStruct((_N,), jnp.float32),
        mesh=mesh,
        compiler_params=pltpu.CompilerParams(needs_layout_passes=False),
        scratch_types=[
            pltpu.VMEM((_NODES,), jnp.float32),
            pltpu.VMEM((_NODES,), jnp.float32),
            pltpu.VMEM((_NODES,), jnp.float32),
            pltpu.VMEM((_NODES,), jnp.float32),
            pltpu.VMEM((_CHUNK,), jnp.float32),
            pltpu.VMEM((_CHUNK,), jnp.float32),
            pltpu.VMEM((_CHUNK,), jnp.float32),
            pltpu.VMEM((_CHUNK,), jnp.float32),
            pltpu.SemaphoreType.DMA,
            pltpu.SemaphoreType.DMA,
            pltpu.SemaphoreType.DMA,
            pltpu.SemaphoreType.DMA,
        ],
    )
    def body(x_hbm, coef_hbm, out_hbm, tab0, tab1, tab2, tab3,
             xin0, xin1, res0, res1, si0, si1, so0, so1):
        cid = lax.axis_index("c")
        sid = lax.axis_index("s")
        wid = sid * _NC + cid
        base = wid * _PER_TILE

        pltpu.sync_copy(coef_hbm.at[0], tab0)
        pltpu.sync_copy(coef_hbm.at[1], tab1)
        pltpu.sync_copy(coef_hbm.at[2], tab2)
        pltpu.sync_copy(coef_hbm.at[3], tab3)

        bufs = ((xin0, res0, si0, so0), (xin1, res1, si1, so1))
        tabs = (tab0, tab1, tab2, tab3)

        def load(c, xin, si):
            return pltpu.make_async_copy(
                x_hbm.at[pl.ds(base + c * _CHUNK, _CHUNK)], xin, si)

        def store(c, res, so):
            return pltpu.make_async_copy(
                res, out_hbm.at[pl.ds(base + c * _CHUNK, _CHUNK)], so)

        load(0, xin0, si0).start()
        load(1, xin1, si1).start()

        def outer(j, _):
            for b in range(2):
                xin, res, si, so = bufs[b]
                c = 2 * j + b
                load(c, xin, si).wait()

                @pl.when(c >= 2)
                def _drain():
                    store(c - 2, res, so).wait()

                @plsc.parallel_loop(0, _CHUNK, step=_LANES, unroll=16)
                def vec(i):
                    xv = xin[pl.ds(i, _LANES)]
                    s = xv * jnp.float32(_NODES - 1)
                    si_ = s.astype(jnp.int32)
                    r = xv - si_.astype(jnp.float32) * jnp.float32(_H)
                    c0 = plsc.load_gather(tabs[0], [si_])
                    c1 = plsc.load_gather(tabs[1], [si_])
                    c2 = plsc.load_gather(tabs[2], [si_])
                    c3 = plsc.load_gather(tabs[3], [si_])
                    res[pl.ds(i, _LANES)] = c0 + r * (c1 + r * (c2 + r * c3))

                store(c, res, so).start()

                @pl.when(c + 2 < _NCHUNK)
                def _next():
                    load(c + 2, xin, si).start()

            return _

        lax.fori_loop(0, _NCHUNK // 2, outer, None)
        store(_NCHUNK - 2, res0, so0).wait()
        store(_NCHUNK - 1, res1, so1).wait()

    return body(x, coef)


@jax.jit
def kernel(input, value):
    coef = _akima_coefs(value)
    return _sc_eval(input, coef)
